# speedup vs baseline: 5.1828x; 5.1828x over previous
import functools

import jax
import jax.numpy as jnp
from jax import lax
from jax.experimental import pallas as pl
from jax.experimental.pallas import tpu as pltpu
from jax.experimental.pallas import tpu_sc as plsc

VOCAB = 100000
D = 128
NC = 2
NS = 16
NW = NC * NS
CHUNK = 128


def _make_gather(n_idx):
    assert n_idx % (NW * CHUNK) == 0
    n_chunks = n_idx // CHUNK
    chunks_per_w = n_chunks // NW
    mesh = plsc.VectorSubcoreMesh(core_axis_name="c", subcore_axis_name="s")

    @functools.partial(
        pl.kernel,
        mesh=mesh,
        out_type=jax.ShapeDtypeStruct((n_chunks, CHUNK, D), jnp.float32),
        scratch_types=[
            pltpu.VMEM((CHUNK,), jnp.int32),
            pltpu.VMEM((CHUNK, D), jnp.float32),
            pltpu.SemaphoreType.DMA,
        ],
    )
    def gather_kernel(idx_hbm, table_hbm, out_hbm, idx_v, rows_v, sem):
        wid = lax.axis_index("s") * NC + lax.axis_index("c")

        def step(i, carry):
            chunk = wid * chunks_per_w + i
            pltpu.sync_copy(idx_hbm.at[pl.ds(chunk * CHUNK, CHUNK)], idx_v)
            pltpu.async_copy(table_hbm.at[idx_v], rows_v, sem).wait()
            pltpu.sync_copy(rows_v, out_hbm.at[chunk])
            return carry

        lax.fori_loop(0, chunks_per_w, step, 0)

    return gather_kernel


def kernel(x, table):
    b, s = x.shape
    idx = x.reshape(-1).astype(jnp.int32)
    out = _make_gather(idx.shape[0])(idx, table)
    return out.reshape(b, s, D)


# speedup vs baseline: 9.2055x; 1.7762x over previous
import functools

import jax
import jax.numpy as jnp
from jax import lax
from jax.experimental import pallas as pl
from jax.experimental.pallas import tpu as pltpu
from jax.experimental.pallas import tpu_sc as plsc

D = 128
NC = 2
NS = 16
NW = NC * NS
CHUNK = 128
K = 2


def _make_gather(n_idx):
    assert n_idx % (NW * CHUNK * K) == 0
    n_chunks = n_idx // CHUNK
    chunks_per_w = n_chunks // NW
    n_groups = chunks_per_w // K
    assert n_groups % 2 == 0 and n_groups >= 4
    half = n_groups // 2
    mesh = plsc.VectorSubcoreMesh(core_axis_name="c", subcore_axis_name="s")

    @functools.partial(
        pl.kernel,
        mesh=mesh,
        out_type=jax.ShapeDtypeStruct((n_chunks, CHUNK, D), jnp.float32),
        scratch_types=[
            pltpu.VMEM((2, K, CHUNK), jnp.int32),
            pltpu.VMEM((2, K, CHUNK, D), jnp.float32),
            pltpu.SemaphoreType.DMA,
            pltpu.SemaphoreType.DMA,
            pltpu.SemaphoreType.DMA,
            pltpu.SemaphoreType.DMA,
            pltpu.SemaphoreType.DMA,
            pltpu.SemaphoreType.DMA,
        ],
    )
    def gather_kernel(idx_hbm, table_hbm, out_hbm, idx_v, rows_v,
                      is0, is1, gs0, gs1, os0, os1):
        isems = (is0, is1)
        gsems = (gs0, gs1)
        osems = (os0, os1)
        wid = lax.axis_index("s") * NC + lax.axis_index("c")
        chunk0 = wid * chunks_per_w

        def idx_load(g, b):
            return pltpu.make_async_copy(
                idx_hbm.at[pl.ds(chunk0 + g * K, K)], idx_v.at[b], isems[b])

        def gather(g, b, j):
            return pltpu.make_async_copy(
                table_hbm.at[idx_v.at[b, j]], rows_v.at[b, j], gsems[b])

        def out_write(g, b):
            return pltpu.make_async_copy(
                rows_v.at[b], out_hbm.at[pl.ds(chunk0 + g * K, K)], osems[b])

        idx_load(0, 0).start()
        idx_load(1, 1).start()
        idx_load(0, 0).wait()
        for j in range(K):
            gather(0, 0, j).start()

        def step(i, carry):
            g = 2 * i
            @pl.when(i >= 1)
            def _():
                out_write(g - 1, 1).wait()
            idx_load(g + 1, 1).wait()
            for j in range(K):
                gather(g + 1, 1, j).start()
            for j in range(K):
                gather(g, 0, j).wait()

            @pl.when(i < half - 1)
            def _():
                idx_load(g + 2, 0).start()
            out_write(g, 0).start()

            g = 2 * i + 1

            @pl.when(i < half - 1)
            def _():
                out_write(g - 1, 0).wait()
                idx_load(g + 1, 0).wait()
                for j in range(K):
                    gather(g + 1, 0, j).start()
            for j in range(K):
                gather(g, 1, j).wait()

            @pl.when(i < half - 1)
            def _():
                idx_load(g + 2, 1).start()
            out_write(g, 1).start()
            return carry

        lax.fori_loop(0, half, step, 0)

        out_write(n_groups - 2, 0).wait()
        out_write(n_groups - 1, 1).wait()

    return gather_kernel


def kernel(x, table):
    b, s = x.shape
    idx = x.reshape(-1, CHUNK).astype(jnp.int32)
    out = _make_gather(idx.size)(idx, table)
    return out.reshape(b, s, D)
